# Initial kernel scaffold; baseline (speedup 1.0000x reference)
import jax
import jax.numpy as jnp
from jax.experimental import pallas as pl


def kernel(x, Wr, Wu_e, Wg_e, Wd_e, Wu_s, Wg_s, Wd_s):
    raise NotImplementedError("write your pallas kernel here")


# speedup vs baseline: 1.7141x; 1.7141x over previous
import functools

import jax
import jax.numpy as jnp
from jax import lax
from jax.experimental import pallas as pl
from jax.experimental.pallas import tpu as pltpu
from jax.experimental.pallas import tpu_sc as plsc

DIM = 768
HID = 1536
NE = 8
ALPHA = 2.0
NTOK = 2048
B = 128
NBMAX = NTOK // B + (NE - 2)
TRASH = NBMAX * B
NALLOC = TRASH + B


def _router_body(x_ref, wr_ref, dst_ref, scale_ref, nb_ref):
    x = x_ref[...]
    wr = wr_ref[...]
    logits = lax.dot_general(x, wr, (((1,), (1,)), ((), ())),
                             preferred_element_type=jnp.float32)
    m = jnp.max(logits, axis=1, keepdims=True)
    e = jnp.exp(logits - m)
    probs = e / jnp.sum(e, axis=1, keepdims=True)
    pmax = jnp.max(probs, axis=1, keepdims=True)
    lanes = lax.broadcasted_iota(jnp.int32, (NTOK, NE), 1)
    idx = jnp.min(jnp.where(probs == pmax, lanes, NE), axis=1, keepdims=True)
    oh = (lanes == idx).astype(jnp.float32)

    rows = lax.broadcasted_iota(jnp.int32, (NTOK, NE), 0)
    r = oh
    k = 1
    while k < NTOK:
        r = r + jnp.where(rows >= k, pltpu.roll(r, k, 0), 0.0)
        k *= 2
    rank = jnp.sum((r - oh) * oh, axis=1, keepdims=True)
    counts = r[NTOK - 1:NTOK, :]
    nb = jnp.floor((counts + (B - 1)) / B)
    is_routed = lax.broadcasted_iota(jnp.int32, (1, NE), 1) < NE - 1
    nb = jnp.where(is_routed, nb, 0.0)
    fi = lax.broadcasted_iota(jnp.int32, (NE, NE), 0)
    gi = lax.broadcasted_iota(jnp.int32, (NE, NE), 1)
    tri = (fi < gi).astype(jnp.float32)
    blkstart = lax.dot_general(nb, tri, (((1,), (0,)), ((), ())),
                               preferred_element_type=jnp.float32)
    off = B * blkstart
    dst = rank + jnp.sum(oh * off, axis=1, keepdims=True)
    dst = jnp.where(idx == NE - 1, float(TRASH), dst)
    dst_ref[...] = dst.astype(jnp.int32)
    scale_ref[...] = jnp.where(idx < NE - 1, ALPHA * pmax, 0.0)
    nb_ref[...] = nb.astype(jnp.int32)


def _router(x, Wr):
    return pl.pallas_call(
        _router_body,
        out_shape=(
            jax.ShapeDtypeStruct((NTOK, 1), jnp.int32),
            jax.ShapeDtypeStruct((NTOK, 1), jnp.float32),
            jax.ShapeDtypeStruct((1, NE), jnp.int32),
        ),
    )(x, Wr)


def _sc_copy_kernel(gather: bool, nrows_out: int):
    info = plsc.get_sparse_core_info()
    nc, ns = info.num_cores, info.num_subcores
    nw = nc * ns
    tpw = NTOK // nw
    mesh = plsc.VectorSubcoreMesh(core_axis_name="c", subcore_axis_name="s")

    @functools.partial(
        pl.kernel,
        out_type=jax.ShapeDtypeStruct((nrows_out, DIM), jnp.float32),
        mesh=mesh,
        scratch_types=[
            pltpu.VMEM((tpw,), jnp.int32),
            pltpu.VMEM((tpw, DIM), jnp.float32),
            pltpu.SemaphoreType.DMA,
        ],
    )
    def k(rows_hbm, dst_hbm, out_hbm, idx_v, rows_v, sem):
        wid = lax.axis_index("s") * nc + lax.axis_index("c")
        base = wid * tpw
        pltpu.sync_copy(dst_hbm.at[pl.ds(base, tpw)], idx_v)
        if gather:
            pltpu.async_copy(rows_hbm.at[idx_v], rows_v, sem).wait()
            pltpu.sync_copy(rows_v, out_hbm.at[pl.ds(base, tpw)])
        else:
            pltpu.sync_copy(rows_hbm.at[pl.ds(base, tpw)], rows_v)
            pltpu.async_copy(rows_v, out_hbm.at[idx_v], sem).wait()

    return k


def _moe_body(s_ref, x_ref, wu_ref, wg_ref, wd_ref, y_ref):
    b = pl.program_id(0)

    @pl.when(b < s_ref[NBMAX])
    def _():
        xx = x_ref[...]
        up = lax.dot_general(xx, wu_ref[0], (((1,), (1,)), ((), ())),
                             preferred_element_type=jnp.float32)
        gt = lax.dot_general(xx, wg_ref[0], (((1,), (1,)), ((), ())),
                             preferred_element_type=jnp.float32)
        h = up * (gt * lax.logistic(gt))
        y_ref[...] = lax.dot_general(h, wd_ref[0], (((1,), (1,)), ((), ())),
                                     preferred_element_type=jnp.float32)

    @pl.when(b >= s_ref[NBMAX])
    def _():
        y_ref[...] = jnp.zeros_like(y_ref)


def _moe_blocks(sinfo, x_pad, Wu_e, Wg_e, Wd_e):
    grid_spec = pltpu.PrefetchScalarGridSpec(
        num_scalar_prefetch=1,
        grid=(NBMAX,),
        in_specs=[
            pl.BlockSpec((B, DIM), lambda b, s: (b, 0)),
            pl.BlockSpec((1, HID, DIM), lambda b, s: (s[b], 0, 0)),
            pl.BlockSpec((1, HID, DIM), lambda b, s: (s[b], 0, 0)),
            pl.BlockSpec((1, DIM, HID), lambda b, s: (s[b], 0, 0)),
        ],
        out_specs=pl.BlockSpec((B, DIM), lambda b, s: (b, 0)),
    )
    return pl.pallas_call(
        _moe_body,
        grid_spec=grid_spec,
        out_shape=jax.ShapeDtypeStruct((NALLOC, DIM), jnp.float32),
    )(sinfo, x_pad, Wu_e, Wg_e, Wd_e)


def _shared_body(x_ref, yr_ref, sc_ref, wu_ref, wg_ref, wd_ref, o_ref):
    xx = x_ref[...]
    up = lax.dot_general(xx, wu_ref[...], (((1,), (1,)), ((), ())),
                         preferred_element_type=jnp.float32)
    gt = lax.dot_general(xx, wg_ref[...], (((1,), (1,)), ((), ())),
                         preferred_element_type=jnp.float32)
    h = up * (gt * lax.logistic(gt))
    sw = lax.dot_general(h, wd_ref[...], (((1,), (1,)), ((), ())),
                         preferred_element_type=jnp.float32)
    s = sc_ref[...]
    o_ref[...] = sw + jnp.where(s > 0, s * yr_ref[...], 0.0)


def _shared_combine(x, y_routed, scale, Wu_s, Wg_s, Wd_s):
    nblk = NTOK // B
    return pl.pallas_call(
        _shared_body,
        grid=(nblk,),
        in_specs=[
            pl.BlockSpec((B, DIM), lambda b: (b, 0)),
            pl.BlockSpec((B, DIM), lambda b: (b, 0)),
            pl.BlockSpec((B, 1), lambda b: (b, 0)),
            pl.BlockSpec((HID, DIM), lambda b: (0, 0)),
            pl.BlockSpec((HID, DIM), lambda b: (0, 0)),
            pl.BlockSpec((DIM, HID), lambda b: (0, 0)),
        ],
        out_specs=pl.BlockSpec((B, DIM), lambda b: (b, 0)),
        out_shape=jax.ShapeDtypeStruct((NTOK, DIM), jnp.float32),
    )(x, y_routed, scale, Wu_s, Wg_s, Wd_s)


def kernel(x, Wr, Wu_e, Wg_e, Wd_e, Wu_s, Wg_s, Wd_s):
    dst2d, scale, nb2d = _router(x, Wr)
    dst = dst2d.reshape(NTOK)

    nbv = nb2d[0, :NE - 1]
    ends = jnp.cumsum(nbv)
    nactive = ends[NE - 2]
    bidx = jnp.arange(NBMAX, dtype=jnp.int32)
    cap = jnp.minimum(bidx, jnp.maximum(nactive - 1, 0))
    be = jnp.sum(cap[:, None] >= ends[None, :], axis=1)
    be = jnp.minimum(be, NE - 2).astype(jnp.int32)
    sinfo = jnp.concatenate([be, nactive[None].astype(jnp.int32)])

    x_pad = _sc_copy_kernel(False, NALLOC)(x, dst)
    y_pad = _moe_blocks(sinfo, x_pad, Wu_e, Wg_e, Wd_e)
    y_routed = _sc_copy_kernel(True, NTOK)(y_pad, dst)
    return _shared_combine(x, y_routed, scale, Wu_s, Wg_s, Wd_s)


# speedup vs baseline: 1.7805x; 1.0387x over previous
import functools

import jax
import jax.numpy as jnp
from jax import lax
from jax.experimental import pallas as pl
from jax.experimental.pallas import tpu as pltpu
from jax.experimental.pallas import tpu_sc as plsc

DIM = 768
HID = 1536
NE = 8
ALPHA = 2.0
NTOK = 2048
B = 128
NBMAX = NTOK // B + (NE - 2)
TRASH = NBMAX * B
NALLOC = TRASH + B


def _router_body(x_ref, wr_ref, dst_ref, scale_ref, nb_ref):
    x = x_ref[...]
    wr = wr_ref[...]
    logits = lax.dot_general(x, wr, (((1,), (1,)), ((), ())),
                             preferred_element_type=jnp.float32)
    m = jnp.max(logits, axis=1, keepdims=True)
    e = jnp.exp(logits - m)
    probs = e / jnp.sum(e, axis=1, keepdims=True)
    pmax = jnp.max(probs, axis=1, keepdims=True)
    lanes = lax.broadcasted_iota(jnp.int32, (NTOK, NE), 1)
    idx = jnp.min(jnp.where(probs == pmax, lanes, NE), axis=1, keepdims=True)
    oh = (lanes == idx).astype(jnp.float32)

    rows = lax.broadcasted_iota(jnp.int32, (NTOK, NE), 0)
    r = oh
    k = 1
    while k < NTOK:
        r = r + jnp.where(rows >= k, pltpu.roll(r, k, 0), 0.0)
        k *= 2
    rank = jnp.sum((r - oh) * oh, axis=1, keepdims=True)
    counts = r[NTOK - 1:NTOK, :]
    nb = jnp.floor((counts + (B - 1)) / B)
    is_routed = lax.broadcasted_iota(jnp.int32, (1, NE), 1) < NE - 1
    nb = jnp.where(is_routed, nb, 0.0)
    fi = lax.broadcasted_iota(jnp.int32, (NE, NE), 0)
    gi = lax.broadcasted_iota(jnp.int32, (NE, NE), 1)
    tri = (fi < gi).astype(jnp.float32)
    blkstart = lax.dot_general(nb, tri, (((1,), (0,)), ((), ())),
                               preferred_element_type=jnp.float32)
    off = B * blkstart
    dst = rank + jnp.sum(oh * off, axis=1, keepdims=True)
    dst = jnp.where(idx == NE - 1, float(TRASH), dst)
    dst_ref[...] = dst.astype(jnp.int32)
    scale_ref[...] = jnp.where(idx < NE - 1, ALPHA * pmax, 0.0)
    nb_ref[...] = nb.astype(jnp.int32)


def _router(x, Wr):
    return pl.pallas_call(
        _router_body,
        out_shape=(
            jax.ShapeDtypeStruct((NTOK, 1), jnp.int32),
            jax.ShapeDtypeStruct((NTOK, 1), jnp.float32),
            jax.ShapeDtypeStruct((1, NE), jnp.int32),
        ),
    )(x, Wr)


def _sc_info():
    info = plsc.get_sparse_core_info()
    nc, ns = info.num_cores, info.num_subcores
    return nc, ns, NTOK // (nc * ns)


def _sc_scatter_kernel():
    nc, _, tpw = _sc_info()
    mesh = plsc.VectorSubcoreMesh(core_axis_name="c", subcore_axis_name="s")

    @functools.partial(
        pl.kernel,
        out_type=jax.ShapeDtypeStruct((NALLOC, DIM), jnp.float32),
        mesh=mesh,
        compiler_params=pltpu.CompilerParams(needs_layout_passes=False),
        scratch_types=[
            pltpu.VMEM((tpw,), jnp.int32),
            pltpu.VMEM((tpw, DIM), jnp.float32),
            pltpu.SemaphoreType.DMA,
        ],
    )
    def k(rows_hbm, dst_hbm, out_hbm, idx_v, rows_v, sem):
        wid = lax.axis_index("s") * nc + lax.axis_index("c")
        base = wid * tpw
        pltpu.sync_copy(dst_hbm.at[pl.ds(base, tpw)], idx_v)
        pltpu.sync_copy(rows_hbm.at[pl.ds(base, tpw)], rows_v)
        pltpu.async_copy(rows_v, out_hbm.at[idx_v], sem).wait()

    return k


def _sc_gather_combine_kernel():
    nc, _, tpw = _sc_info()
    mesh = plsc.VectorSubcoreMesh(core_axis_name="c", subcore_axis_name="s")

    @functools.partial(
        pl.kernel,
        out_type=jax.ShapeDtypeStruct((NTOK, DIM), jnp.float32),
        mesh=mesh,
        compiler_params=pltpu.CompilerParams(needs_layout_passes=False),
        scratch_types=[
            pltpu.VMEM((tpw,), jnp.int32),
            pltpu.VMEM((tpw,), jnp.float32),
            pltpu.VMEM((tpw, DIM), jnp.float32),
            pltpu.VMEM((tpw, DIM), jnp.float32),
            pltpu.SemaphoreType.DMA,
        ],
    )
    def k(ypad_hbm, dst_hbm, sh_hbm, sc_hbm, out_hbm,
          idx_v, sc_v, rows_v, acc_v, sem):
        wid = lax.axis_index("s") * nc + lax.axis_index("c")
        base = wid * tpw
        pltpu.sync_copy(dst_hbm.at[pl.ds(base, tpw)], idx_v)
        pltpu.sync_copy(sc_hbm.at[pl.ds(base, tpw)], sc_v)
        pltpu.sync_copy(sh_hbm.at[pl.ds(base, tpw)], acc_v)
        pltpu.async_copy(ypad_hbm.at[idx_v], rows_v, sem).wait()

        lane = lax.iota(jnp.int32, 16)

        def body(t, carry):
            s16 = sc_v[pl.ds((t // 16) * 16, 16)]
            s = jnp.sum(jnp.where(lane == (t % 16), s16, 0.0))
            sv = jnp.full((16,), s)
            use = sv > 0.0
            for j in range(DIM // 16):
                sl = pl.ds(j * 16, 16)
                r = jnp.where(use, sv * rows_v[t, sl], 0.0)
                acc_v[t, sl] = acc_v[t, sl] + r
            return carry

        lax.fori_loop(0, tpw, body, 0)
        pltpu.sync_copy(acc_v, out_hbm.at[pl.ds(base, tpw)])

    return k


def _moe_body(s_ref, x_ref, wu_ref, wg_ref, wd_ref, y_ref):
    b = pl.program_id(0)

    @pl.when(b < s_ref[NBMAX])
    def _():
        xx = x_ref[...]
        up = lax.dot_general(xx, wu_ref[0], (((1,), (1,)), ((), ())),
                             preferred_element_type=jnp.float32)
        gt = lax.dot_general(xx, wg_ref[0], (((1,), (1,)), ((), ())),
                             preferred_element_type=jnp.float32)
        h = up * (gt * lax.logistic(gt))
        y_ref[...] = lax.dot_general(h, wd_ref[0], (((1,), (1,)), ((), ())),
                                     preferred_element_type=jnp.float32)

    @pl.when(b >= s_ref[NBMAX])
    def _():
        y_ref[...] = jnp.zeros_like(y_ref)


def _moe_blocks(sinfo, x_pad, Wu_e, Wg_e, Wd_e):
    grid_spec = pltpu.PrefetchScalarGridSpec(
        num_scalar_prefetch=1,
        grid=(NBMAX,),
        in_specs=[
            pl.BlockSpec((B, DIM), lambda b, s: (b, 0)),
            pl.BlockSpec((1, HID, DIM), lambda b, s: (s[b], 0, 0)),
            pl.BlockSpec((1, HID, DIM), lambda b, s: (s[b], 0, 0)),
            pl.BlockSpec((1, DIM, HID), lambda b, s: (s[b], 0, 0)),
        ],
        out_specs=pl.BlockSpec((B, DIM), lambda b, s: (b, 0)),
    )
    return pl.pallas_call(
        _moe_body,
        grid_spec=grid_spec,
        out_shape=jax.ShapeDtypeStruct((NALLOC, DIM), jnp.float32),
    )(sinfo, x_pad, Wu_e, Wg_e, Wd_e)


def _shared_body(x_ref, wu_ref, wg_ref, wd_ref, o_ref):
    xx = x_ref[...]
    up = lax.dot_general(xx, wu_ref[...], (((1,), (1,)), ((), ())),
                         preferred_element_type=jnp.float32)
    gt = lax.dot_general(xx, wg_ref[...], (((1,), (1,)), ((), ())),
                         preferred_element_type=jnp.float32)
    h = up * (gt * lax.logistic(gt))
    o_ref[...] = lax.dot_general(h, wd_ref[...], (((1,), (1,)), ((), ())),
                                 preferred_element_type=jnp.float32)


def _shared_swiglu(x, Wu_s, Wg_s, Wd_s):
    nblk = NTOK // B
    return pl.pallas_call(
        _shared_body,
        grid=(nblk,),
        in_specs=[
            pl.BlockSpec((B, DIM), lambda b: (b, 0)),
            pl.BlockSpec((HID, DIM), lambda b: (0, 0)),
            pl.BlockSpec((HID, DIM), lambda b: (0, 0)),
            pl.BlockSpec((DIM, HID), lambda b: (0, 0)),
        ],
        out_specs=pl.BlockSpec((B, DIM), lambda b: (b, 0)),
        out_shape=jax.ShapeDtypeStruct((NTOK, DIM), jnp.float32),
    )(x, Wu_s, Wg_s, Wd_s)


def kernel(x, Wr, Wu_e, Wg_e, Wd_e, Wu_s, Wg_s, Wd_s):
    dst2d, scale2d, nb2d = _router(x, Wr)
    dst = dst2d.reshape(NTOK)
    scale = scale2d.reshape(NTOK)

    nbv = nb2d[0, :NE - 1]
    ends = jnp.cumsum(nbv)
    nactive = ends[NE - 2]
    bidx = jnp.arange(NBMAX, dtype=jnp.int32)
    cap = jnp.minimum(bidx, jnp.maximum(nactive - 1, 0))
    be = jnp.sum(cap[:, None] >= ends[None, :], axis=1)
    be = jnp.minimum(be, NE - 2).astype(jnp.int32)
    sinfo = jnp.concatenate([be, nactive[None].astype(jnp.int32)])

    shared = _shared_swiglu(x, Wu_s, Wg_s, Wd_s)
    x_pad = _sc_scatter_kernel()(x, dst)
    y_pad = _moe_blocks(sinfo, x_pad, Wu_e, Wg_e, Wd_e)
    return _sc_gather_combine_kernel()(y_pad, dst, shared, scale)


# speedup vs baseline: 1.7824x; 1.0011x over previous
import functools

import jax
import jax.numpy as jnp
from jax import lax
from jax.experimental import pallas as pl
from jax.experimental.pallas import tpu as pltpu
from jax.experimental.pallas import tpu_sc as plsc

DIM = 768
HID = 1536
NE = 8
ALPHA = 2.0
NTOK = 2048
B = 128
NBMAX = NTOK // B + (NE - 2)
TRASH = NBMAX * B
NALLOC = TRASH + B


def _router_body(x_ref, wr_ref, dst_ref, scale_ref, nb_ref):
    x = x_ref[...]
    wr = wr_ref[...]
    logits = lax.dot_general(x, wr, (((1,), (1,)), ((), ())),
                             preferred_element_type=jnp.float32)
    m = jnp.max(logits, axis=1, keepdims=True)
    e = jnp.exp(logits - m)
    probs = e / jnp.sum(e, axis=1, keepdims=True)
    pmax = jnp.max(probs, axis=1, keepdims=True)
    lanes = lax.broadcasted_iota(jnp.int32, (NTOK, NE), 1)
    idx = jnp.min(jnp.where(probs == pmax, lanes, NE), axis=1, keepdims=True)
    oh = (lanes == idx).astype(jnp.float32)

    rows = lax.broadcasted_iota(jnp.int32, (NTOK, NE), 0)
    r = oh
    k = 1
    while k < NTOK:
        r = r + jnp.where(rows >= k, pltpu.roll(r, k, 0), 0.0)
        k *= 2
    rank = jnp.sum((r - oh) * oh, axis=1, keepdims=True)
    counts = r[NTOK - 1:NTOK, :]
    nb = jnp.floor((counts + (B - 1)) / B)
    is_routed = lax.broadcasted_iota(jnp.int32, (1, NE), 1) < NE - 1
    nb = jnp.where(is_routed, nb, 0.0)
    fi = lax.broadcasted_iota(jnp.int32, (NE, NE), 0)
    gi = lax.broadcasted_iota(jnp.int32, (NE, NE), 1)
    tri = (fi < gi).astype(jnp.float32)
    blkstart = lax.dot_general(nb, tri, (((1,), (0,)), ((), ())),
                               preferred_element_type=jnp.float32)
    off = B * blkstart
    dst = rank + jnp.sum(oh * off, axis=1, keepdims=True)
    dst = jnp.where(idx == NE - 1, float(TRASH), dst)
    dst_ref[...] = dst.astype(jnp.int32)
    scale_ref[...] = jnp.where(idx < NE - 1, ALPHA * pmax, 0.0)
    nb_ref[...] = nb.astype(jnp.int32)


def _router(x, Wr):
    return pl.pallas_call(
        _router_body,
        out_shape=(
            jax.ShapeDtypeStruct((NTOK, 1), jnp.int32),
            jax.ShapeDtypeStruct((NTOK, 1), jnp.float32),
            jax.ShapeDtypeStruct((1, NE), jnp.int32),
        ),
    )(x, Wr)


def _sc_info():
    info = plsc.get_sparse_core_info()
    nc, ns = info.num_cores, info.num_subcores
    return nc, ns, NTOK // (nc * ns)


def _sc_scatter_kernel():
    nc, _, tpw = _sc_info()
    mesh = plsc.VectorSubcoreMesh(core_axis_name="c", subcore_axis_name="s")

    @functools.partial(
        pl.kernel,
        out_type=jax.ShapeDtypeStruct((NALLOC, DIM), jnp.float32),
        mesh=mesh,
        compiler_params=pltpu.CompilerParams(needs_layout_passes=False),
        scratch_types=[
            pltpu.VMEM((tpw,), jnp.int32),
            pltpu.VMEM((tpw, DIM), jnp.float32),
            pltpu.SemaphoreType.DMA,
        ],
    )
    def k(rows_hbm, dst_hbm, out_hbm, idx_v, rows_v, sem):
        wid = lax.axis_index("s") * nc + lax.axis_index("c")
        base = wid * tpw
        pltpu.sync_copy(dst_hbm.at[pl.ds(base, tpw)], idx_v)
        pltpu.sync_copy(rows_hbm.at[pl.ds(base, tpw)], rows_v)
        pltpu.async_copy(rows_v, out_hbm.at[idx_v], sem).wait()

    return k


def _sc_gather_combine_kernel():
    nc, _, tpw = _sc_info()
    mesh = plsc.VectorSubcoreMesh(core_axis_name="c", subcore_axis_name="s")

    @functools.partial(
        pl.kernel,
        out_type=jax.ShapeDtypeStruct((NTOK, DIM), jnp.float32),
        mesh=mesh,
        compiler_params=pltpu.CompilerParams(needs_layout_passes=False),
        scratch_types=[
            pltpu.VMEM((tpw,), jnp.int32),
            pltpu.VMEM((tpw,), jnp.float32),
            pltpu.VMEM((tpw, DIM), jnp.float32),
            pltpu.VMEM((tpw, DIM), jnp.float32),
            pltpu.SemaphoreType.DMA,
        ],
    )
    def k(ypad_hbm, dst_hbm, sh_hbm, sc_hbm, out_hbm,
          idx_v, sc_v, rows_v, acc_v, sem):
        wid = lax.axis_index("s") * nc + lax.axis_index("c")
        base = wid * tpw
        pltpu.sync_copy(dst_hbm.at[pl.ds(base, tpw)], idx_v)
        pltpu.sync_copy(sc_hbm.at[pl.ds(base, tpw)], sc_v)
        pltpu.sync_copy(sh_hbm.at[pl.ds(base, tpw)], acc_v)
        pltpu.async_copy(ypad_hbm.at[idx_v], rows_v, sem).wait()

        lane = lax.iota(jnp.int32, 16)

        def body(t, carry):
            s16 = sc_v[pl.ds((t // 16) * 16, 16)]
            s = jnp.sum(jnp.where(lane == (t % 16), s16, 0.0))
            sv = jnp.full((16,), s)
            use = sv > 0.0
            for j in range(DIM // 16):
                sl = pl.ds(j * 16, 16)
                r = jnp.where(use, sv * rows_v[t, sl], 0.0)
                acc_v[t, sl] = acc_v[t, sl] + r
            return carry

        lax.fori_loop(0, tpw, body, 0)
        pltpu.sync_copy(acc_v, out_hbm.at[pl.ds(base, tpw)])

    return k


def _moe_body(s_ref, x_ref, wu_ref, wg_ref, wd_ref, y_ref):
    b = pl.program_id(0)

    @pl.when(b < s_ref[NBMAX])
    def _():
        xx = x_ref[...].astype(jnp.bfloat16)
        up = lax.dot_general(xx, wu_ref[0].astype(jnp.bfloat16),
                             (((1,), (1,)), ((), ())),
                             preferred_element_type=jnp.float32)
        gt = lax.dot_general(xx, wg_ref[0].astype(jnp.bfloat16),
                             (((1,), (1,)), ((), ())),
                             preferred_element_type=jnp.float32)
        h = (up * (gt * lax.logistic(gt))).astype(jnp.bfloat16)
        y_ref[...] = lax.dot_general(h, wd_ref[0].astype(jnp.bfloat16),
                                     (((1,), (1,)), ((), ())),
                                     preferred_element_type=jnp.float32)

    @pl.when(b >= s_ref[NBMAX])
    def _():
        y_ref[...] = jnp.zeros_like(y_ref)


def _moe_blocks(sinfo, x_pad, Wu_e, Wg_e, Wd_e):
    grid_spec = pltpu.PrefetchScalarGridSpec(
        num_scalar_prefetch=1,
        grid=(NBMAX,),
        in_specs=[
            pl.BlockSpec((B, DIM), lambda b, s: (b, 0)),
            pl.BlockSpec((1, HID, DIM), lambda b, s: (s[b], 0, 0)),
            pl.BlockSpec((1, HID, DIM), lambda b, s: (s[b], 0, 0)),
            pl.BlockSpec((1, DIM, HID), lambda b, s: (s[b], 0, 0)),
        ],
        out_specs=pl.BlockSpec((B, DIM), lambda b, s: (b, 0)),
    )
    return pl.pallas_call(
        _moe_body,
        grid_spec=grid_spec,
        out_shape=jax.ShapeDtypeStruct((NALLOC, DIM), jnp.float32),
    )(sinfo, x_pad, Wu_e, Wg_e, Wd_e)


def _shared_body(x_ref, wu_ref, wg_ref, wd_ref, o_ref):
    xx = x_ref[...].astype(jnp.bfloat16)
    up = lax.dot_general(xx, wu_ref[...].astype(jnp.bfloat16),
                         (((1,), (1,)), ((), ())),
                         preferred_element_type=jnp.float32)
    gt = lax.dot_general(xx, wg_ref[...].astype(jnp.bfloat16),
                         (((1,), (1,)), ((), ())),
                         preferred_element_type=jnp.float32)
    h = (up * (gt * lax.logistic(gt))).astype(jnp.bfloat16)
    o_ref[...] = lax.dot_general(h, wd_ref[...].astype(jnp.bfloat16),
                                 (((1,), (1,)), ((), ())),
                                 preferred_element_type=jnp.float32)


def _shared_swiglu(x, Wu_s, Wg_s, Wd_s):
    nblk = NTOK // B
    return pl.pallas_call(
        _shared_body,
        grid=(nblk,),
        in_specs=[
            pl.BlockSpec((B, DIM), lambda b: (b, 0)),
            pl.BlockSpec((HID, DIM), lambda b: (0, 0)),
            pl.BlockSpec((HID, DIM), lambda b: (0, 0)),
            pl.BlockSpec((DIM, HID), lambda b: (0, 0)),
        ],
        out_specs=pl.BlockSpec((B, DIM), lambda b: (b, 0)),
        out_shape=jax.ShapeDtypeStruct((NTOK, DIM), jnp.float32),
    )(x, Wu_s, Wg_s, Wd_s)


def kernel(x, Wr, Wu_e, Wg_e, Wd_e, Wu_s, Wg_s, Wd_s):
    dst2d, scale2d, nb2d = _router(x, Wr)
    dst = dst2d.reshape(NTOK)
    scale = scale2d.reshape(NTOK)

    nbv = nb2d[0, :NE - 1]
    ends = jnp.cumsum(nbv)
    nactive = ends[NE - 2]
    bidx = jnp.arange(NBMAX, dtype=jnp.int32)
    cap = jnp.minimum(bidx, jnp.maximum(nactive - 1, 0))
    be = jnp.sum(cap[:, None] >= ends[None, :], axis=1)
    be = jnp.minimum(be, NE - 2).astype(jnp.int32)
    sinfo = jnp.concatenate([be, nactive[None].astype(jnp.int32)])

    shared = _shared_swiglu(x, Wu_s, Wg_s, Wd_s)
    x_pad = _sc_scatter_kernel()(x, dst)
    y_pad = _moe_blocks(sinfo, x_pad, Wu_e, Wg_e, Wd_e)
    return _sc_gather_combine_kernel()(y_pad, dst, shared, scale)


# speedup vs baseline: 2.0982x; 1.1772x over previous
import functools

import jax
import jax.numpy as jnp
from jax import lax
from jax.experimental import pallas as pl
from jax.experimental.pallas import tpu as pltpu
from jax.experimental.pallas import tpu_sc as plsc

DIM = 768
HID = 1536
NE = 8
ALPHA = 2.0
NTOK = 2048
B = 128
NBMAX = NTOK // B + (NE - 2)
TRASH = NBMAX * B
NALLOC = TRASH + B


def _router_body(x_ref, wr_ref, dst_ref, scale_ref, sinfo_ref):
    x = x_ref[...]
    wr = wr_ref[...]
    logits = lax.dot_general(x, wr, (((1,), (1,)), ((), ())),
                             preferred_element_type=jnp.float32)
    m = jnp.max(logits, axis=1, keepdims=True)
    e = jnp.exp(logits - m)
    probs = e / jnp.sum(e, axis=1, keepdims=True)
    pmax = jnp.max(probs, axis=1, keepdims=True)
    lanes = lax.broadcasted_iota(jnp.int32, (NTOK, NE), 1)
    idx = jnp.min(jnp.where(probs == pmax, lanes, NE), axis=1, keepdims=True)
    oh = (lanes == idx).astype(jnp.float32)

    rows = lax.broadcasted_iota(jnp.int32, (NTOK, NE), 0)
    r = oh
    k = 1
    while k < NTOK:
        r = r + jnp.where(rows >= k, pltpu.roll(r, k, 0), 0.0)
        k *= 2
    rank = jnp.sum((r - oh) * oh, axis=1, keepdims=True)
    counts = r[NTOK - 1:NTOK, :]
    nb = jnp.floor((counts + (B - 1)) / B)
    is_routed = lax.broadcasted_iota(jnp.int32, (1, NE), 1) < NE - 1
    nb = jnp.where(is_routed, nb, 0.0)
    fi = lax.broadcasted_iota(jnp.int32, (NE, NE), 0)
    gi = lax.broadcasted_iota(jnp.int32, (NE, NE), 1)
    tri = (fi < gi).astype(jnp.float32)
    blkstart = lax.dot_general(nb, tri, (((1,), (0,)), ((), ())),
                               preferred_element_type=jnp.float32)
    off = B * blkstart
    dst = rank + jnp.sum(oh * off, axis=1, keepdims=True)
    dst = jnp.where(idx == NE - 1, float(TRASH), dst)
    dst_ref[...] = dst.astype(jnp.int32)
    scale_ref[...] = jnp.where(idx < NE - 1, ALPHA * pmax, 0.0)

    ends = lax.dot_general(nb, (fi <= gi).astype(jnp.float32),
                           (((1,), (0,)), ((), ())),
                           preferred_element_type=jnp.float32)
    nactive = jnp.max(ends)
    bcol = lax.broadcasted_iota(jnp.int32, (NBMAX + 1, 1), 0).astype(jnp.float32)
    cap = jnp.minimum(bcol, jnp.maximum(nactive - 1.0, 0.0))
    be = jnp.sum((cap >= ends).astype(jnp.float32), axis=1, keepdims=True)
    be = jnp.minimum(be, NE - 2)
    sinfo = jnp.where(bcol >= NBMAX, nactive, be)
    sinfo_ref[...] = sinfo.astype(jnp.int32)


def _router(x, Wr):
    return pl.pallas_call(
        _router_body,
        out_shape=(
            jax.ShapeDtypeStruct((NTOK, 1), jnp.int32),
            jax.ShapeDtypeStruct((NTOK, 1), jnp.float32),
            jax.ShapeDtypeStruct((NBMAX + 1, 1), jnp.int32),
        ),
    )(x, Wr)


def _sc_info():
    info = plsc.get_sparse_core_info()
    nc, ns = info.num_cores, info.num_subcores
    return nc, ns, NTOK // (nc * ns)


def _sc_scatter_kernel():
    nc, _, tpw = _sc_info()
    mesh = plsc.VectorSubcoreMesh(core_axis_name="c", subcore_axis_name="s")

    @functools.partial(
        pl.kernel,
        out_type=jax.ShapeDtypeStruct((NALLOC, DIM), jnp.float32),
        mesh=mesh,
        compiler_params=pltpu.CompilerParams(needs_layout_passes=False),
        scratch_types=[
            pltpu.VMEM((tpw,), jnp.int32),
            pltpu.VMEM((tpw, DIM), jnp.float32),
            pltpu.SemaphoreType.DMA,
        ],
    )
    def k(rows_hbm, dst_hbm, out_hbm, idx_v, rows_v, sem):
        wid = lax.axis_index("s") * nc + lax.axis_index("c")
        base = wid * tpw
        pltpu.sync_copy(dst_hbm.at[pl.ds(base, tpw)], idx_v)
        pltpu.sync_copy(rows_hbm.at[pl.ds(base, tpw)], rows_v)
        pltpu.async_copy(rows_v, out_hbm.at[idx_v], sem).wait()

    return k


def _sc_gather_combine_kernel():
    nc, _, tpw = _sc_info()
    mesh = plsc.VectorSubcoreMesh(core_axis_name="c", subcore_axis_name="s")

    @functools.partial(
        pl.kernel,
        out_type=jax.ShapeDtypeStruct((NTOK, DIM), jnp.float32),
        mesh=mesh,
        compiler_params=pltpu.CompilerParams(needs_layout_passes=False),
        scratch_types=[
            pltpu.VMEM((tpw,), jnp.int32),
            pltpu.VMEM((tpw,), jnp.float32),
            pltpu.VMEM((tpw, DIM), jnp.float32),
            pltpu.VMEM((tpw, DIM), jnp.float32),
            pltpu.SemaphoreType.DMA,
        ],
    )
    def k(ypad_hbm, dst_hbm, sh_hbm, sc_hbm, out_hbm,
          idx_v, sc_v, rows_v, acc_v, sem):
        wid = lax.axis_index("s") * nc + lax.axis_index("c")
        base = wid * tpw
        pltpu.sync_copy(dst_hbm.at[pl.ds(base, tpw)], idx_v)
        pltpu.sync_copy(sc_hbm.at[pl.ds(base, tpw)], sc_v)
        pltpu.sync_copy(sh_hbm.at[pl.ds(base, tpw)], acc_v)
        pltpu.async_copy(ypad_hbm.at[idx_v], rows_v, sem).wait()

        lane = lax.iota(jnp.int32, 16)

        def body(t, carry):
            s16 = sc_v[pl.ds((t // 16) * 16, 16)]
            s = jnp.sum(jnp.where(lane == (t % 16), s16, 0.0))
            sv = jnp.full((16,), s)
            use = sv > 0.0
            for j in range(DIM // 16):
                sl = pl.ds(j * 16, 16)
                r = jnp.where(use, sv * rows_v[t, sl], 0.0)
                acc_v[t, sl] = acc_v[t, sl] + r
            return carry

        lax.fori_loop(0, tpw, body, 0)
        pltpu.sync_copy(acc_v, out_hbm.at[pl.ds(base, tpw)])

    return k


def _moe_body(s_ref, x_ref, wu_ref, wg_ref, wd_ref, y_ref):
    b = pl.program_id(0)

    @pl.when(b < s_ref[NBMAX])
    def _():
        xx = x_ref[...].astype(jnp.bfloat16)
        up = lax.dot_general(xx, wu_ref[0].astype(jnp.bfloat16),
                             (((1,), (1,)), ((), ())),
                             preferred_element_type=jnp.float32)
        gt = lax.dot_general(xx, wg_ref[0].astype(jnp.bfloat16),
                             (((1,), (1,)), ((), ())),
                             preferred_element_type=jnp.float32)
        h = (up * (gt * lax.logistic(gt))).astype(jnp.bfloat16)
        y_ref[...] = lax.dot_general(h, wd_ref[0].astype(jnp.bfloat16),
                                     (((1,), (1,)), ((), ())),
                                     preferred_element_type=jnp.float32)

    @pl.when(b >= s_ref[NBMAX])
    def _():
        y_ref[...] = jnp.zeros_like(y_ref)


def _moe_blocks(sinfo, x_pad, Wu_e, Wg_e, Wd_e):
    grid_spec = pltpu.PrefetchScalarGridSpec(
        num_scalar_prefetch=1,
        grid=(NBMAX,),
        in_specs=[
            pl.BlockSpec((B, DIM), lambda b, s: (b, 0)),
            pl.BlockSpec((1, HID, DIM), lambda b, s: (s[b], 0, 0)),
            pl.BlockSpec((1, HID, DIM), lambda b, s: (s[b], 0, 0)),
            pl.BlockSpec((1, DIM, HID), lambda b, s: (s[b], 0, 0)),
        ],
        out_specs=pl.BlockSpec((B, DIM), lambda b, s: (b, 0)),
    )
    return pl.pallas_call(
        _moe_body,
        grid_spec=grid_spec,
        out_shape=jax.ShapeDtypeStruct((NALLOC, DIM), jnp.float32),
    )(sinfo, x_pad, Wu_e, Wg_e, Wd_e)


def _shared_body(x_ref, wu_ref, wg_ref, wd_ref, o_ref):
    xx = x_ref[...].astype(jnp.bfloat16)
    up = lax.dot_general(xx, wu_ref[...].astype(jnp.bfloat16),
                         (((1,), (1,)), ((), ())),
                         preferred_element_type=jnp.float32)
    gt = lax.dot_general(xx, wg_ref[...].astype(jnp.bfloat16),
                         (((1,), (1,)), ((), ())),
                         preferred_element_type=jnp.float32)
    h = (up * (gt * lax.logistic(gt))).astype(jnp.bfloat16)
    o_ref[...] = lax.dot_general(h, wd_ref[...].astype(jnp.bfloat16),
                                 (((1,), (1,)), ((), ())),
                                 preferred_element_type=jnp.float32)


BSH = 512


def _shared_swiglu(x, Wu_s, Wg_s, Wd_s):
    nblk = NTOK // BSH
    return pl.pallas_call(
        _shared_body,
        grid=(nblk,),
        in_specs=[
            pl.BlockSpec((BSH, DIM), lambda b: (b, 0)),
            pl.BlockSpec((HID, DIM), lambda b: (0, 0)),
            pl.BlockSpec((HID, DIM), lambda b: (0, 0)),
            pl.BlockSpec((DIM, HID), lambda b: (0, 0)),
        ],
        out_specs=pl.BlockSpec((BSH, DIM), lambda b: (b, 0)),
        out_shape=jax.ShapeDtypeStruct((NTOK, DIM), jnp.float32),
    )(x, Wu_s, Wg_s, Wd_s)


def kernel(x, Wr, Wu_e, Wg_e, Wd_e, Wu_s, Wg_s, Wd_s):
    dst2d, scale2d, sinfo2d = _router(x, Wr)
    dst = dst2d.reshape(NTOK)
    scale = scale2d.reshape(NTOK)
    sinfo = sinfo2d.reshape(NBMAX + 1)

    shared = _shared_swiglu(x, Wu_s, Wg_s, Wd_s)
    x_pad = _sc_scatter_kernel()(x, dst)
    y_pad = _moe_blocks(sinfo, x_pad, Wu_e, Wg_e, Wd_e)
    return _sc_gather_combine_kernel()(y_pad, dst, shared, scale)


# speedup vs baseline: 2.4572x; 1.1711x over previous
import functools

import jax
import jax.numpy as jnp
from jax import lax
from jax.experimental import pallas as pl
from jax.experimental.pallas import tpu as pltpu
from jax.experimental.pallas import tpu_sc as plsc

DIM = 768
HID = 1536
NE = 8
ALPHA = 2.0
NTOK = 2048
B = 256
NBMAX = NTOK // B + (NE - 2)
TRASH = NBMAX * B
NALLOC = TRASH + B


def _router_body(x_ref, wr_ref, dst_ref, scale_ref, sinfo_ref):
    x = x_ref[...]
    wr = wr_ref[...]
    logits = lax.dot_general(x, wr, (((1,), (1,)), ((), ())),
                             preferred_element_type=jnp.float32)
    m = jnp.max(logits, axis=1, keepdims=True)
    e = jnp.exp(logits - m)
    probs = e / jnp.sum(e, axis=1, keepdims=True)
    pmax = jnp.max(probs, axis=1, keepdims=True)
    lanes = lax.broadcasted_iota(jnp.int32, (NTOK, NE), 1)
    idx = jnp.min(jnp.where(probs == pmax, lanes, NE), axis=1, keepdims=True)
    oh = (lanes == idx).astype(jnp.float32)

    rows = lax.broadcasted_iota(jnp.int32, (NTOK, NE), 0)
    r = oh
    k = 1
    while k < NTOK:
        r = r + jnp.where(rows >= k, pltpu.roll(r, k, 0), 0.0)
        k *= 2
    rank = jnp.sum((r - oh) * oh, axis=1, keepdims=True)
    counts = r[NTOK - 1:NTOK, :]
    nb = jnp.floor((counts + (B - 1)) / B)
    is_routed = lax.broadcasted_iota(jnp.int32, (1, NE), 1) < NE - 1
    nb = jnp.where(is_routed, nb, 0.0)
    fi = lax.broadcasted_iota(jnp.int32, (NE, NE), 0)
    gi = lax.broadcasted_iota(jnp.int32, (NE, NE), 1)
    tri = (fi < gi).astype(jnp.float32)
    blkstart = lax.dot_general(nb, tri, (((1,), (0,)), ((), ())),
                               preferred_element_type=jnp.float32)
    off = B * blkstart
    dst = rank + jnp.sum(oh * off, axis=1, keepdims=True)
    dst = jnp.where(idx == NE - 1, float(TRASH), dst)
    dst_ref[...] = dst.astype(jnp.int32)
    scale_ref[...] = jnp.where(idx < NE - 1, ALPHA * pmax, 0.0)

    ends = lax.dot_general(nb, (fi <= gi).astype(jnp.float32),
                           (((1,), (0,)), ((), ())),
                           preferred_element_type=jnp.float32)
    nactive = jnp.max(ends)
    bcol = lax.broadcasted_iota(jnp.int32, (NBMAX + 1, 1), 0).astype(jnp.float32)
    cap = jnp.minimum(bcol, jnp.maximum(nactive - 1.0, 0.0))
    be = jnp.sum((cap >= ends).astype(jnp.float32), axis=1, keepdims=True)
    be = jnp.minimum(be, NE - 2)
    sinfo = jnp.where(bcol >= NBMAX, nactive, be)
    sinfo_ref[...] = sinfo.astype(jnp.int32)


def _router(x, Wr):
    return pl.pallas_call(
        _router_body,
        out_shape=(
            jax.ShapeDtypeStruct((NTOK, 1), jnp.int32),
            jax.ShapeDtypeStruct((NTOK, 1), jnp.float32),
            jax.ShapeDtypeStruct((NBMAX + 1, 1), jnp.int32),
        ),
    )(x, Wr)


def _sc_info():
    info = plsc.get_sparse_core_info()
    nc, ns = info.num_cores, info.num_subcores
    return nc, ns, NTOK // (nc * ns)


def _sc_scatter_kernel():
    nc, _, tpw = _sc_info()
    mesh = plsc.VectorSubcoreMesh(core_axis_name="c", subcore_axis_name="s")

    @functools.partial(
        pl.kernel,
        out_type=jax.ShapeDtypeStruct((NALLOC, DIM), jnp.float32),
        mesh=mesh,
        compiler_params=pltpu.CompilerParams(needs_layout_passes=False),
        scratch_types=[
            pltpu.VMEM((tpw,), jnp.int32),
            pltpu.VMEM((tpw, DIM), jnp.float32),
            pltpu.SemaphoreType.DMA,
        ],
    )
    def k(rows_hbm, dst_hbm, out_hbm, idx_v, rows_v, sem):
        wid = lax.axis_index("s") * nc + lax.axis_index("c")
        base = wid * tpw
        pltpu.sync_copy(dst_hbm.at[pl.ds(base, tpw)], idx_v)
        pltpu.sync_copy(rows_hbm.at[pl.ds(base, tpw)], rows_v)
        pltpu.async_copy(rows_v, out_hbm.at[idx_v], sem).wait()

    return k


def _sc_gather_combine_kernel():
    nc, _, tpw = _sc_info()
    mesh = plsc.VectorSubcoreMesh(core_axis_name="c", subcore_axis_name="s")

    @functools.partial(
        pl.kernel,
        out_type=jax.ShapeDtypeStruct((NTOK, DIM), jnp.float32),
        mesh=mesh,
        compiler_params=pltpu.CompilerParams(needs_layout_passes=False),
        scratch_types=[
            pltpu.VMEM((tpw,), jnp.int32),
            pltpu.VMEM((tpw,), jnp.float32),
            pltpu.VMEM((tpw, DIM), jnp.float32),
            pltpu.VMEM((tpw, DIM), jnp.float32),
            pltpu.SemaphoreType.DMA,
        ],
    )
    def k(ypad_hbm, dst_hbm, sh_hbm, sc_hbm, out_hbm,
          idx_v, sc_v, rows_v, acc_v, sem):
        wid = lax.axis_index("s") * nc + lax.axis_index("c")
        base = wid * tpw
        pltpu.sync_copy(dst_hbm.at[pl.ds(base, tpw)], idx_v)
        pltpu.sync_copy(sc_hbm.at[pl.ds(base, tpw)], sc_v)
        pltpu.sync_copy(sh_hbm.at[pl.ds(base, tpw)], acc_v)
        pltpu.async_copy(ypad_hbm.at[idx_v], rows_v, sem).wait()

        lane = lax.iota(jnp.int32, 16)

        def body(t, carry):
            s16 = sc_v[pl.ds((t // 16) * 16, 16)]
            s = jnp.sum(jnp.where(lane == (t % 16), s16, 0.0))
            sv = jnp.full((16,), s)
            use = sv > 0.0
            for j in range(DIM // 16):
                sl = pl.ds(j * 16, 16)
                r = jnp.where(use, sv * rows_v[t, sl], 0.0)
                acc_v[t, sl] = acc_v[t, sl] + r
            return carry

        lax.fori_loop(0, tpw, body, 0)
        pltpu.sync_copy(acc_v, out_hbm.at[pl.ds(base, tpw)])

    return k


def _moe_body(s_ref, x_ref, wu_ref, wg_ref, wd_ref, y_ref):
    b = pl.program_id(0)

    @pl.when(b < s_ref[NBMAX])
    def _():
        xx = x_ref[...].astype(jnp.bfloat16)
        up = lax.dot_general(xx, wu_ref[0].astype(jnp.bfloat16),
                             (((1,), (1,)), ((), ())),
                             preferred_element_type=jnp.float32)
        gt = lax.dot_general(xx, wg_ref[0].astype(jnp.bfloat16),
                             (((1,), (1,)), ((), ())),
                             preferred_element_type=jnp.float32)
        h = (up * (gt * lax.logistic(gt))).astype(jnp.bfloat16)
        y_ref[...] = lax.dot_general(h, wd_ref[0].astype(jnp.bfloat16),
                                     (((1,), (1,)), ((), ())),
                                     preferred_element_type=jnp.float32)

    @pl.when(b >= s_ref[NBMAX])
    def _():
        y_ref[...] = jnp.zeros_like(y_ref)


def _moe_blocks(sinfo, x_pad, Wu_e, Wg_e, Wd_e):
    grid_spec = pltpu.PrefetchScalarGridSpec(
        num_scalar_prefetch=1,
        grid=(NBMAX,),
        in_specs=[
            pl.BlockSpec((B, DIM), lambda b, s: (b, 0)),
            pl.BlockSpec((1, HID, DIM), lambda b, s: (s[b], 0, 0)),
            pl.BlockSpec((1, HID, DIM), lambda b, s: (s[b], 0, 0)),
            pl.BlockSpec((1, DIM, HID), lambda b, s: (s[b], 0, 0)),
        ],
        out_specs=pl.BlockSpec((B, DIM), lambda b, s: (b, 0)),
    )
    return pl.pallas_call(
        _moe_body,
        grid_spec=grid_spec,
        out_shape=jax.ShapeDtypeStruct((NALLOC, DIM), jnp.float32),
    )(sinfo, x_pad, Wu_e, Wg_e, Wd_e)


def _shared_body(x_ref, wu_ref, wg_ref, wd_ref, o_ref):
    xx = x_ref[...].astype(jnp.bfloat16)
    up = lax.dot_general(xx, wu_ref[...].astype(jnp.bfloat16),
                         (((1,), (1,)), ((), ())),
                         preferred_element_type=jnp.float32)
    gt = lax.dot_general(xx, wg_ref[...].astype(jnp.bfloat16),
                         (((1,), (1,)), ((), ())),
                         preferred_element_type=jnp.float32)
    h = (up * (gt * lax.logistic(gt))).astype(jnp.bfloat16)
    o_ref[...] = lax.dot_general(h, wd_ref[...].astype(jnp.bfloat16),
                                 (((1,), (1,)), ((), ())),
                                 preferred_element_type=jnp.float32)


BSH = 512


def _shared_swiglu(x, Wu_s, Wg_s, Wd_s):
    nblk = NTOK // BSH
    return pl.pallas_call(
        _shared_body,
        grid=(nblk,),
        in_specs=[
            pl.BlockSpec((BSH, DIM), lambda b: (b, 0)),
            pl.BlockSpec((HID, DIM), lambda b: (0, 0)),
            pl.BlockSpec((HID, DIM), lambda b: (0, 0)),
            pl.BlockSpec((DIM, HID), lambda b: (0, 0)),
        ],
        out_specs=pl.BlockSpec((BSH, DIM), lambda b: (b, 0)),
        out_shape=jax.ShapeDtypeStruct((NTOK, DIM), jnp.float32),
    )(x, Wu_s, Wg_s, Wd_s)


def kernel(x, Wr, Wu_e, Wg_e, Wd_e, Wu_s, Wg_s, Wd_s):
    dst2d, scale2d, sinfo2d = _router(x, Wr)
    dst = dst2d.reshape(NTOK)
    scale = scale2d.reshape(NTOK)
    sinfo = sinfo2d.reshape(NBMAX + 1)

    shared = _shared_swiglu(x, Wu_s, Wg_s, Wd_s)
    x_pad = _sc_scatter_kernel()(x, dst)
    y_pad = _moe_blocks(sinfo, x_pad, Wu_e, Wg_e, Wd_e)
    return _sc_gather_combine_kernel()(y_pad, dst, shared, scale)


# speedup vs baseline: 2.5977x; 1.0572x over previous
import functools

import jax
import jax.numpy as jnp
from jax import lax
from jax.experimental import pallas as pl
from jax.experimental.pallas import tpu as pltpu
from jax.experimental.pallas import tpu_sc as plsc

DIM = 768
HID = 1536
NE = 8
ALPHA = 2.0
NTOK = 2048
B = 512
NBMAX = NTOK // B + (NE - 2)
TRASH = NBMAX * B
NALLOC = TRASH + B


def _router_body(x_ref, wr_ref, dst_ref, scale_ref, sinfo_ref):
    x = x_ref[...]
    wr = wr_ref[...]
    logits = lax.dot_general(x, wr, (((1,), (1,)), ((), ())),
                             preferred_element_type=jnp.float32)
    m = jnp.max(logits, axis=1, keepdims=True)
    e = jnp.exp(logits - m)
    probs = e / jnp.sum(e, axis=1, keepdims=True)
    pmax = jnp.max(probs, axis=1, keepdims=True)
    lanes = lax.broadcasted_iota(jnp.int32, (NTOK, NE), 1)
    idx = jnp.min(jnp.where(probs == pmax, lanes, NE), axis=1, keepdims=True)
    oh = (lanes == idx).astype(jnp.float32)

    rows = lax.broadcasted_iota(jnp.int32, (NTOK, NE), 0)
    r = oh
    k = 1
    while k < NTOK:
        r = r + jnp.where(rows >= k, pltpu.roll(r, k, 0), 0.0)
        k *= 2
    rank = jnp.sum((r - oh) * oh, axis=1, keepdims=True)
    counts = r[NTOK - 1:NTOK, :]
    nb = jnp.floor((counts + (B - 1)) / B)
    is_routed = lax.broadcasted_iota(jnp.int32, (1, NE), 1) < NE - 1
    nb = jnp.where(is_routed, nb, 0.0)
    fi = lax.broadcasted_iota(jnp.int32, (NE, NE), 0)
    gi = lax.broadcasted_iota(jnp.int32, (NE, NE), 1)
    tri = (fi < gi).astype(jnp.float32)
    blkstart = lax.dot_general(nb, tri, (((1,), (0,)), ((), ())),
                               preferred_element_type=jnp.float32)
    off = B * blkstart
    dst = rank + jnp.sum(oh * off, axis=1, keepdims=True)
    dst = jnp.where(idx == NE - 1, float(TRASH), dst)
    dst_ref[...] = dst.astype(jnp.int32)
    scale_ref[...] = jnp.where(idx < NE - 1, ALPHA * pmax, 0.0)

    ends = lax.dot_general(nb, (fi <= gi).astype(jnp.float32),
                           (((1,), (0,)), ((), ())),
                           preferred_element_type=jnp.float32)
    nactive = jnp.max(ends)
    bcol = lax.broadcasted_iota(jnp.int32, (NBMAX + 1, 1), 0).astype(jnp.float32)
    cap = jnp.minimum(bcol, jnp.maximum(nactive - 1.0, 0.0))
    be = jnp.sum((cap >= ends).astype(jnp.float32), axis=1, keepdims=True)
    be = jnp.minimum(be, NE - 2)
    sinfo = jnp.where(bcol >= NBMAX, nactive, be)
    sinfo_ref[...] = sinfo.astype(jnp.int32)


def _router(x, Wr):
    return pl.pallas_call(
        _router_body,
        out_shape=(
            jax.ShapeDtypeStruct((NTOK, 1), jnp.int32),
            jax.ShapeDtypeStruct((NTOK, 1), jnp.float32),
            jax.ShapeDtypeStruct((NBMAX + 1, 1), jnp.int32),
        ),
    )(x, Wr)


def _sc_info():
    info = plsc.get_sparse_core_info()
    nc, ns = info.num_cores, info.num_subcores
    return nc, ns, NTOK // (nc * ns)


def _sc_scatter_kernel():
    nc, _, tpw = _sc_info()
    mesh = plsc.VectorSubcoreMesh(core_axis_name="c", subcore_axis_name="s")

    @functools.partial(
        pl.kernel,
        out_type=jax.ShapeDtypeStruct((NALLOC, DIM), jnp.float32),
        mesh=mesh,
        compiler_params=pltpu.CompilerParams(needs_layout_passes=False),
        scratch_types=[
            pltpu.VMEM((tpw,), jnp.int32),
            pltpu.VMEM((tpw, DIM), jnp.float32),
            pltpu.SemaphoreType.DMA,
        ],
    )
    def k(rows_hbm, dst_hbm, out_hbm, idx_v, rows_v, sem):
        wid = lax.axis_index("s") * nc + lax.axis_index("c")
        base = wid * tpw
        pltpu.sync_copy(dst_hbm.at[pl.ds(base, tpw)], idx_v)
        pltpu.sync_copy(rows_hbm.at[pl.ds(base, tpw)], rows_v)
        pltpu.async_copy(rows_v, out_hbm.at[idx_v], sem).wait()

    return k


def _sc_gather_combine_kernel():
    nc, _, tpw = _sc_info()
    mesh = plsc.VectorSubcoreMesh(core_axis_name="c", subcore_axis_name="s")

    @functools.partial(
        pl.kernel,
        out_type=jax.ShapeDtypeStruct((NTOK, DIM), jnp.float32),
        mesh=mesh,
        compiler_params=pltpu.CompilerParams(needs_layout_passes=False),
        scratch_types=[
            pltpu.VMEM((tpw,), jnp.int32),
            pltpu.VMEM((tpw,), jnp.float32),
            pltpu.VMEM((tpw, DIM), jnp.float32),
            pltpu.VMEM((tpw, DIM), jnp.float32),
            pltpu.SemaphoreType.DMA,
        ],
    )
    def k(ypad_hbm, dst_hbm, sh_hbm, sc_hbm, out_hbm,
          idx_v, sc_v, rows_v, acc_v, sem):
        wid = lax.axis_index("s") * nc + lax.axis_index("c")
        base = wid * tpw
        pltpu.sync_copy(dst_hbm.at[pl.ds(base, tpw)], idx_v)
        pltpu.sync_copy(sc_hbm.at[pl.ds(base, tpw)], sc_v)
        pltpu.sync_copy(sh_hbm.at[pl.ds(base, tpw)], acc_v)
        pltpu.async_copy(ypad_hbm.at[idx_v], rows_v, sem).wait()

        lane = lax.iota(jnp.int32, 16)

        def body(t, carry):
            s16 = sc_v[pl.ds((t // 16) * 16, 16)]
            s = jnp.sum(jnp.where(lane == (t % 16), s16, 0.0))
            sv = jnp.full((16,), s)
            use = sv > 0.0
            for j in range(DIM // 16):
                sl = pl.ds(j * 16, 16)
                r = jnp.where(use, sv * rows_v[t, sl], 0.0)
                acc_v[t, sl] = acc_v[t, sl] + r
            return carry

        lax.fori_loop(0, tpw, body, 0)
        pltpu.sync_copy(acc_v, out_hbm.at[pl.ds(base, tpw)])

    return k


def _moe_body(s_ref, x_ref, wu_ref, wg_ref, wd_ref, y_ref):
    b = pl.program_id(0)

    @pl.when(b < s_ref[NBMAX])
    def _():
        xx = x_ref[...].astype(jnp.bfloat16)
        up = lax.dot_general(xx, wu_ref[0].astype(jnp.bfloat16),
                             (((1,), (1,)), ((), ())),
                             preferred_element_type=jnp.float32)
        gt = lax.dot_general(xx, wg_ref[0].astype(jnp.bfloat16),
                             (((1,), (1,)), ((), ())),
                             preferred_element_type=jnp.float32)
        h = (up * (gt * lax.logistic(gt))).astype(jnp.bfloat16)
        y_ref[...] = lax.dot_general(h, wd_ref[0].astype(jnp.bfloat16),
                                     (((1,), (1,)), ((), ())),
                                     preferred_element_type=jnp.float32)

    @pl.when(b >= s_ref[NBMAX])
    def _():
        y_ref[...] = jnp.zeros_like(y_ref)


def _moe_blocks(sinfo, x_pad, Wu_e, Wg_e, Wd_e):
    grid_spec = pltpu.PrefetchScalarGridSpec(
        num_scalar_prefetch=1,
        grid=(NBMAX,),
        in_specs=[
            pl.BlockSpec((B, DIM), lambda b, s: (b, 0)),
            pl.BlockSpec((1, HID, DIM), lambda b, s: (s[b], 0, 0)),
            pl.BlockSpec((1, HID, DIM), lambda b, s: (s[b], 0, 0)),
            pl.BlockSpec((1, DIM, HID), lambda b, s: (s[b], 0, 0)),
        ],
        out_specs=pl.BlockSpec((B, DIM), lambda b, s: (b, 0)),
    )
    return pl.pallas_call(
        _moe_body,
        grid_spec=grid_spec,
        out_shape=jax.ShapeDtypeStruct((NALLOC, DIM), jnp.float32),
    )(sinfo, x_pad, Wu_e, Wg_e, Wd_e)


def _shared_body(x_ref, wu_ref, wg_ref, wd_ref, o_ref):
    xx = x_ref[...].astype(jnp.bfloat16)
    up = lax.dot_general(xx, wu_ref[...].astype(jnp.bfloat16),
                         (((1,), (1,)), ((), ())),
                         preferred_element_type=jnp.float32)
    gt = lax.dot_general(xx, wg_ref[...].astype(jnp.bfloat16),
                         (((1,), (1,)), ((), ())),
                         preferred_element_type=jnp.float32)
    h = (up * (gt * lax.logistic(gt))).astype(jnp.bfloat16)
    o_ref[...] = lax.dot_general(h, wd_ref[...].astype(jnp.bfloat16),
                                 (((1,), (1,)), ((), ())),
                                 preferred_element_type=jnp.float32)


BSH = 1024


def _shared_swiglu(x, Wu_s, Wg_s, Wd_s):
    nblk = NTOK // BSH
    return pl.pallas_call(
        _shared_body,
        grid=(nblk,),
        in_specs=[
            pl.BlockSpec((BSH, DIM), lambda b: (b, 0)),
            pl.BlockSpec((HID, DIM), lambda b: (0, 0)),
            pl.BlockSpec((HID, DIM), lambda b: (0, 0)),
            pl.BlockSpec((DIM, HID), lambda b: (0, 0)),
        ],
        out_specs=pl.BlockSpec((BSH, DIM), lambda b: (b, 0)),
        out_shape=jax.ShapeDtypeStruct((NTOK, DIM), jnp.float32),
    )(x, Wu_s, Wg_s, Wd_s)


def kernel(x, Wr, Wu_e, Wg_e, Wd_e, Wu_s, Wg_s, Wd_s):
    dst2d, scale2d, sinfo2d = _router(x, Wr)
    dst = dst2d.reshape(NTOK)
    scale = scale2d.reshape(NTOK)
    sinfo = sinfo2d.reshape(NBMAX + 1)

    shared = _shared_swiglu(x, Wu_s, Wg_s, Wd_s)
    x_pad = _sc_scatter_kernel()(x, dst)
    y_pad = _moe_blocks(sinfo, x_pad, Wu_e, Wg_e, Wd_e)
    return _sc_gather_combine_kernel()(y_pad, dst, shared, scale)
